# baseline (speedup 1.0000x reference)
import functools

import jax
import jax.numpy as jnp
from jax import lax
from jax.experimental import pallas as pl
from jax.experimental.pallas import tpu as pltpu
from jax.experimental.pallas import tpu_sc as plsc

_D = 128
_NW = 32
_NBUF = 8


@functools.lru_cache(maxsize=None)
def _build(batch: int, hist: int, vocab: int):
    b_per_w = batch // _NW
    assert batch % _NW == 0 and b_per_w % _NBUF == 0 and hist <= 128

    mesh = plsc.VectorSubcoreMesh(core_axis_name="c", subcore_axis_name="s")

    @functools.partial(
        pl.kernel,
        out_type=jax.ShapeDtypeStruct((batch, hist, _D), jnp.float32),
        mesh=mesh,
        scratch_types=[
            pltpu.VMEM((b_per_w, hist), jnp.int32),
            pltpu.VMEM((_NBUF, hist, _D), jnp.float32),
        ] + [pltpu.SemaphoreType.DMA] * (2 * _NBUF),
        compiler_params=pltpu.CompilerParams(use_tc_tiling_on_sc=True),
    )
    def gather_kernel(idx_hbm, table_hbm, out_hbm, idx_v, rows_v, *sems):
        gsems = sems[:_NBUF]
        osems = sems[_NBUF:]
        wid = lax.axis_index("s") * 2 + lax.axis_index("c")
        base = wid * b_per_w

        pltpu.sync_copy(idx_hbm.at[pl.ds(base, b_per_w)], idx_v)

        for b in range(_NBUF):
            pltpu.async_copy(table_hbm.at[idx_v.at[b]], rows_v.at[b], gsems[b])

        @pl.loop(_NBUF, b_per_w, step=_NBUF)
        def _(j0):
            for b in range(_NBUF):
                j = j0 + b
                pltpu.make_async_copy(
                    table_hbm.at[idx_v.at[b]], rows_v.at[b], gsems[b]
                ).wait()
                dst = out_hbm.at[base + j - _NBUF]
                pltpu.async_copy(rows_v.at[b], dst, osems[b])
                pltpu.make_async_copy(rows_v.at[b], dst, osems[b]).wait()
                pltpu.async_copy(table_hbm.at[idx_v.at[j]], rows_v.at[b], gsems[b])

        for b in range(_NBUF):
            j = b_per_w - _NBUF + b
            pltpu.make_async_copy(
                table_hbm.at[idx_v.at[b]], rows_v.at[b], gsems[b]
            ).wait()
            pltpu.sync_copy(rows_v.at[b], out_hbm.at[base + j])

    return gather_kernel


@jax.jit
def kernel(indices, table):
    batch, hist = indices.shape
    vocab, dim = table.shape
    assert dim == _D
    fn = _build(batch, hist, vocab)
    return fn(indices.astype(jnp.int32), table)


# speedup vs baseline: 1.0017x; 1.0017x over previous
import functools

import jax
import jax.numpy as jnp
from jax import lax
from jax.experimental import pallas as pl
from jax.experimental.pallas import tpu as pltpu
from jax.experimental.pallas import tpu_sc as plsc

_D = 128
_NW = 32
_NBUF = 8


@functools.lru_cache(maxsize=None)
def _build(batch: int, hist: int, vocab: int):
    b_per_w = batch // _NW
    assert batch % _NW == 0 and b_per_w % _NBUF == 0 and hist <= 128

    mesh = plsc.VectorSubcoreMesh(core_axis_name="c", subcore_axis_name="s")

    @functools.partial(
        pl.kernel,
        out_type=jax.ShapeDtypeStruct((batch, hist, _D), jnp.float32),
        mesh=mesh,
        scratch_types=[
            pltpu.VMEM((b_per_w, hist), jnp.int32),
            pltpu.VMEM((_NBUF, hist, _D), jnp.float32),
        ] + [pltpu.SemaphoreType.DMA] * (2 * _NBUF),
        compiler_params=pltpu.CompilerParams(
            use_tc_tiling_on_sc=True, skip_device_barrier=True
        ),
    )
    def gather_kernel(idx_hbm, table_hbm, out_hbm, idx_v, rows_v, *sems):
        gsems = sems[:_NBUF]
        osems = sems[_NBUF:]
        wid = lax.axis_index("s") * 2 + lax.axis_index("c")
        base = wid * b_per_w

        pltpu.sync_copy(idx_hbm.at[pl.ds(base, b_per_w)], idx_v)

        for b in range(_NBUF):
            pltpu.async_copy(table_hbm.at[idx_v.at[b]], rows_v.at[b], gsems[b])

        @pl.loop(_NBUF, b_per_w, step=_NBUF)
        def _(j0):
            for b in range(_NBUF):
                j = j0 + b
                pltpu.make_async_copy(
                    table_hbm.at[idx_v.at[b]], rows_v.at[b], gsems[b]
                ).wait()
                dst = out_hbm.at[base + j - _NBUF]
                pltpu.async_copy(rows_v.at[b], dst, osems[b])
                pltpu.make_async_copy(rows_v.at[b], dst, osems[b]).wait()
                pltpu.async_copy(table_hbm.at[idx_v.at[j]], rows_v.at[b], gsems[b])

        for b in range(_NBUF):
            j = b_per_w - _NBUF + b
            pltpu.make_async_copy(
                table_hbm.at[idx_v.at[b]], rows_v.at[b], gsems[b]
            ).wait()
            pltpu.sync_copy(rows_v.at[b], out_hbm.at[base + j])

    return gather_kernel


@jax.jit
def kernel(indices, table):
    batch, hist = indices.shape
    vocab, dim = table.shape
    assert dim == _D
    fn = _build(batch, hist, vocab)
    return fn(indices.astype(jnp.int32), table)
